# baseline (speedup 1.0000x reference)
import functools

import jax
import jax.numpy as jnp
from jax import lax
from jax.experimental import pallas as pl
from jax.experimental.pallas import tpu as pltpu

_NUM_CLASSES = 21
_BG = 20
_RATIO = 3
_N = 131072
_R = 64
_LANES = 128
_C = _R * _LANES
_NB = _N // _C


def _tc_body(cats_ref, bbs_ref, gtb_ref, gt_ref, out_ref,
             amain_ref, aloc_ref, anp_ref, cnts_ref):
    p = pl.program_id(0)
    j = pl.program_id(1)

    g = gt_ref[...]
    negb = jnp.where(g == _BG, 1.0, 0.0)
    posf = 1.0 - negb

    @pl.when(p == 0)
    def _phase0():
        @pl.when(j == 0)
        def _init():
            amain_ref[...] = jnp.zeros_like(amain_ref)
            aloc_ref[...] = jnp.zeros_like(aloc_ref)
            anp_ref[...] = jnp.zeros_like(anp_ref)

        cnts_ref[j] = jnp.sum(negb)

    @pl.when(p == 1)
    def _phase1():
        def pbody(w, carry):
            base, tot = carry
            cw = cnts_ref[w]
            return (base + jnp.where(w < j, cw, 0.0), tot + cw)

        base, totneg = lax.fori_loop(0, _NB, pbody, (0.0, 0.0))
        kf = jnp.float32(_RATIO) * (jnp.float32(_N) - totneg)

        x = cats_ref[...]
        sp = jnp.maximum(x, 0.0) + jnp.log1p(jnp.exp(-jnp.abs(x)))
        colsum = jnp.sum(sp, axis=0) - sp[_BG]
        row = lax.broadcasted_iota(jnp.int32, x.shape, 0)
        gtp = jnp.where(g == _BG, _NUM_CLASSES + 2, g)
        xc = jnp.sum(jnp.where(row == gtp[None], x, 0.0), axis=0)

        li = lax.broadcasted_iota(jnp.int32, (_LANES, _LANES), 0)
        ci = lax.broadcasted_iota(jnp.int32, (_LANES, _LANES), 1)
        upper = jnp.where(li <= ci, 1.0, 0.0)
        pin = jax.lax.dot_general(negb, upper, (((1,), (0,)), ((), ())),
                                  preferred_element_type=jnp.float32)
        ri = lax.broadcasted_iota(jnp.int32, (_R, _R), 0)
        cj = lax.broadcasted_iota(jnp.int32, (_R, _R), 1)
        lstrict = jnp.where(cj < ri, 1.0, 0.0)
        rowtot = pin[:, _LANES - 1:_LANES]
        rowoff = jax.lax.dot_general(lstrict, rowtot, (((1,), (0,)), ((), ())),
                                     preferred_element_type=jnp.float32)
        incl = pin + rowoff
        selw = jnp.where((negb > 0.5) & (base + incl <= kf), 1.0, 0.0)
        nsp = colsum - posf * colsum

        d = bbs_ref[...] - gtb_ref[...]
        ad = jnp.abs(d)
        l1 = jnp.where(ad < 1.0, 0.5 * d * d, ad - 0.5)
        locs = jnp.sum(l1, axis=0) * posf

        amain_ref[...] += posf * colsum - xc + selw * nsp
        aloc_ref[...] += locs
        anp_ref[...] += posf

        @pl.when(j == pl.num_programs(1) - 1)
        def _fini():
            n = jnp.sum(anp_ref[...])
            conf = jnp.sum(amain_ref[...])
            loc = jnp.sum(aloc_ref[...])
            out_ref[0] = (conf + loc) / n
            out_ref[1] = loc
            out_ref[2] = conf


def _tc_loss(cats3, bbs3, gtb3, gt2):
    return pl.pallas_call(
        _tc_body,
        grid=(2, _NB),
        in_specs=[
            pl.BlockSpec((_NUM_CLASSES, _R, _LANES), lambda p, j: (0, j * p, 0)),
            pl.BlockSpec((4, _R, _LANES), lambda p, j: (0, j * p, 0)),
            pl.BlockSpec((4, _R, _LANES), lambda p, j: (0, j * p, 0)),
            pl.BlockSpec((_R, _LANES), lambda p, j: (j, 0)),
        ],
        out_specs=pl.BlockSpec(memory_space=pltpu.SMEM),
        out_shape=jax.ShapeDtypeStruct((3,), jnp.float32),
        scratch_shapes=[
            pltpu.VMEM((_R, _LANES), jnp.float32),
            pltpu.VMEM((_R, _LANES), jnp.float32),
            pltpu.VMEM((_R, _LANES), jnp.float32),
            pltpu.SMEM((_NB,), jnp.float32),
        ],
    )(cats3, bbs3, gtb3, gt2)


def kernel(bbs_preds, cats_preds, gt_bbs, gt_cats):
    gt = gt_cats.astype(jnp.int32)
    out = _tc_loss(
        cats_preds.T.reshape(_NUM_CLASSES, _N // _LANES, _LANES),
        bbs_preds.T.reshape(4, _N // _LANES, _LANES),
        gt_bbs.T.reshape(4, _N // _LANES, _LANES),
        gt.reshape(_N // _LANES, _LANES),
    )
    return (out[0], out[1], out[2])


# speedup vs baseline: 2.0569x; 2.0569x over previous
import functools

import jax
import jax.numpy as jnp
from jax import lax
from jax.experimental import pallas as pl
from jax.experimental.pallas import tpu as pltpu

_NUM_CLASSES = 21
_BG = 20
_RATIO = 3
_N = 131072
_R = 64
_LANES = 128
_C = _R * _LANES
_NB = _N // _C


def _tc_body(cats_ref, bbs_ref, gtb_ref, gt_ref, gtr_ref, out_ref,
             amain_ref, aloc_ref, anp_ref, cnts_ref):
    p = pl.program_id(0)
    j = pl.program_id(1)

    gr = gtr_ref[...]
    negb = jnp.where(gr == _BG, 1.0, 0.0)
    posf64 = 1.0 - negb

    @pl.when(p == 0)
    def _phase0():
        @pl.when(j == 0)
        def _init():
            amain_ref[...] = jnp.zeros_like(amain_ref)
            aloc_ref[...] = jnp.zeros_like(aloc_ref)
            anp_ref[...] = jnp.zeros_like(anp_ref)

        cnts_ref[j] = jnp.sum(negb)

    @pl.when(p == 1)
    def _phase1():
        def pbody(w, carry):
            base, tot = carry
            cw = cnts_ref[w]
            return (base + jnp.where(w < j, cw, 0.0), tot + cw)

        base, totneg = lax.fori_loop(0, _NB, pbody, (0.0, 0.0))
        kf = jnp.float32(_RATIO) * (jnp.float32(_N) - totneg)

        x = cats_ref[...]
        gt = gt_ref[...]
        posf = jnp.where(gt != _BG, 1.0, 0.0)
        sp = jnp.maximum(x, 0.0) + jnp.log1p(jnp.exp(-jnp.abs(x)))
        colsum = jnp.sum(sp, axis=0, keepdims=True) - sp[_BG:_BG + 1]
        row = lax.broadcasted_iota(jnp.int32, x.shape, 0)
        gtp = jnp.where(gt == _BG, _NUM_CLASSES + 2, gt)
        xc = jnp.sum(jnp.where(row == gtp, x, 0.0), axis=0, keepdims=True)
        nsp = colsum - posf * colsum
        nsp64 = nsp.reshape(_R, _LANES)

        li = lax.broadcasted_iota(jnp.int32, (_LANES, _LANES), 0)
        ci = lax.broadcasted_iota(jnp.int32, (_LANES, _LANES), 1)
        upper = jnp.where(li <= ci, 1.0, 0.0)
        pin = jax.lax.dot_general(negb, upper, (((1,), (0,)), ((), ())),
                                  preferred_element_type=jnp.float32)
        ri = lax.broadcasted_iota(jnp.int32, (_R, _R), 0)
        cj = lax.broadcasted_iota(jnp.int32, (_R, _R), 1)
        lstrict = jnp.where(cj < ri, 1.0, 0.0)
        rowtot = pin[:, _LANES - 1:_LANES]
        rowoff = jax.lax.dot_general(lstrict, rowtot,
                                     (((1,), (0,)), ((), ())),
                                     preferred_element_type=jnp.float32)
        incl = pin + rowoff
        selw = jnp.where((negb > 0.5) & (base + incl <= kf), 1.0, 0.0)

        d = bbs_ref[...] - gtb_ref[...]
        ad = jnp.abs(d)
        l1 = jnp.where(ad < 1.0, 0.5 * d * d, ad - 0.5)
        locs = jnp.sum(l1, axis=0, keepdims=True) * posf

        amain_ref[...] += posf * colsum - xc
        aloc_ref[...] += locs
        anp_ref[...] += posf
        mined = jnp.sum(selw * nsp64, axis=0, keepdims=True)
        amain_ref[0:1, 0:_LANES] += mined

        @pl.when(j == pl.num_programs(1) - 1)
        def _fini():
            n = jnp.sum(anp_ref[...])
            conf = jnp.sum(amain_ref[...])
            loc = jnp.sum(aloc_ref[...])
            out_ref[0] = (conf + loc) / n
            out_ref[1] = loc
            out_ref[2] = conf


def _tc_loss(catsT, bbsT, gtbT, gt1, gtr):
    return pl.pallas_call(
        _tc_body,
        grid=(2, _NB),
        in_specs=[
            pl.BlockSpec((_NUM_CLASSES, _C), lambda p, j: (0, j * p)),
            pl.BlockSpec((4, _C), lambda p, j: (0, j * p)),
            pl.BlockSpec((4, _C), lambda p, j: (0, j * p)),
            pl.BlockSpec((1, _C), lambda p, j: (0, j * p)),
            pl.BlockSpec((_R, _LANES), lambda p, j: (j, 0)),
        ],
        out_specs=pl.BlockSpec(memory_space=pltpu.SMEM),
        out_shape=jax.ShapeDtypeStruct((3,), jnp.float32),
        scratch_shapes=[
            pltpu.VMEM((1, _C), jnp.float32),
            pltpu.VMEM((1, _C), jnp.float32),
            pltpu.VMEM((1, _C), jnp.float32),
            pltpu.SMEM((_NB,), jnp.float32),
        ],
    )(catsT, bbsT, gtbT, gt1, gtr)


def kernel(bbs_preds, cats_preds, gt_bbs, gt_cats):
    gt = gt_cats.astype(jnp.int32)
    out = _tc_loss(
        cats_preds.T,
        bbs_preds.T,
        gt_bbs.T,
        gt.reshape(1, _N),
        gt.reshape(_N // _LANES, _LANES),
    )
    return (out[0], out[1], out[2])
